# baseline (speedup 1.0000x reference)
import functools

import jax
import jax.numpy as jnp
from jax import lax
from jax.experimental import pallas as pl
from jax.experimental.pallas import tpu as pltpu

N = 10000
NHID = 128
K = 64
TEMP = 1.0

_HBLK = 2000


def _h_body(x_ref, w_ref, o_ref):
    o_ref[...] = jnp.dot(x_ref[...], w_ref[...], preferred_element_type=jnp.float32)


def _linear(x, W):
    return pl.pallas_call(
        _h_body,
        grid=(N // _HBLK,),
        in_specs=[
            pl.BlockSpec((_HBLK, NHID), lambda i: (i, 0)),
            pl.BlockSpec((NHID, NHID), lambda i: (0, 0)),
        ],
        out_specs=pl.BlockSpec((_HBLK, NHID), lambda i: (i, 0)),
        out_shape=jax.ShapeDtypeStruct((N, NHID), jnp.float32),
    )(x, W)


_R = 8


def _medoid_body(hsel_ref, a_ref, rs_ref, bias_ref, o_ref):
    xb = hsel_ref[...]
    gram = jnp.dot(xb, xb.T, preferred_element_type=jnp.float32)
    sqall = jnp.sum(xb * xb, axis=1)
    bias = bias_ref[...]
    for g in range(_R):
        inner = gram[g * K:(g + 1) * K, g * K:(g + 1) * K]
        sq = sqall[g * K:(g + 1) * K]
        a = a_ref[g, :]
        rs = rs_ref[g, 0]
        d2 = sq[:, None] + sq[None, :] - 2.0 * inner
        dist = jnp.sqrt(jnp.maximum(d2, 0.0) + 1e-12)
        c = jnp.sum(dist * a[None, :], axis=1)
        valid = a > 0
        cmin = jnp.min(jnp.where(valid, c, 1e30))
        w = jnp.exp(-(c - cmin) / (TEMP * rs)) * a
        w = w / jnp.maximum(jnp.sum(w), 1e-12)
        xs = xb[g * K:(g + 1) * K, :]
        out = rs * jnp.sum(w[:, None] * xs, axis=0, keepdims=True)
        o_ref[g, :] = jnp.maximum(out + bias, 0.0)[0]


def _medoid(hsel2d, a, rs2d, bias2d):
    return pl.pallas_call(
        _medoid_body,
        grid=(N // _R,),
        in_specs=[
            pl.BlockSpec((_R * K, NHID), lambda i: (i, 0)),
            pl.BlockSpec((_R, K), lambda i: (i, 0)),
            pl.BlockSpec((_R, 1), lambda i: (i, 0)),
            pl.BlockSpec((1, NHID), lambda i: (0, 0)),
        ],
        out_specs=pl.BlockSpec((_R, NHID), lambda i: (i, 0)),
        out_shape=jax.ShapeDtypeStruct((N, NHID), jnp.float32),
    )(hsel2d, a, rs2d, bias2d)


_BM = 400


def _gram_body(hi_ref, hj_ref, o_ref):
    o_ref[...] = lax.dot_general(
        hi_ref[...], hj_ref[...],
        dimension_numbers=(((1,), (1,)), ((), ())),
        preferred_element_type=jnp.float32,
    )


def _gram(hout):
    nb = N // _BM
    return pl.pallas_call(
        _gram_body,
        grid=(nb,),
        in_specs=[
            pl.BlockSpec((_BM, NHID), lambda i: (i, 0)),
            pl.BlockSpec((N, NHID), lambda i: (0, 0)),
        ],
        out_specs=pl.BlockSpec((_BM, N), lambda i: (i, 0)),
        out_shape=jax.ShapeDtypeStruct((N, N), jnp.float32),
    )(hout, hout)


def kernel(x, edge_index, W, bias):
    row, col = edge_index[0], edge_index[1]
    loops = jnp.arange(N, dtype=row.dtype)
    rowf = jnp.concatenate([row, loops])
    colf = jnp.concatenate([col, loops])
    deg = jnp.zeros((N,), jnp.float32).at[colf].add(1.0)
    dis = lax.rsqrt(deg)
    norm = dis[rowf] * dis[colf]
    A = jnp.zeros((N, N), jnp.float32).at[rowf, colf].add(norm)
    a, idx = lax.top_k(A, K)
    rsum = A.sum(-1)

    h = _linear(x, W)
    hsel = h[idx.reshape(-1)]
    hout = _medoid(hsel, a, rsum.reshape(N, 1), bias.reshape(1, NHID))
    return _gram(hout)
